# Initial kernel scaffold; baseline (speedup 1.0000x reference)
import functools

import jax
import jax.numpy as jnp
from jax import lax
from jax.experimental import pallas as pl
from jax.experimental.pallas import tpu as pltpu
from jax.experimental.pallas import tpu_sc as plsc

N_NODES = 10000
FEAT = 128
NC, NS, LANES = 2, 16, 16
NW = NC * NS
CHUNK = 128


def _spmm_sc(features, src_r, dst_r, w_r, n_chunks):
    mesh = plsc.VectorSubcoreMesh(core_axis_name="c", subcore_axis_name="s")
    rows_per_sub = N_NODES // NS

    @functools.partial(
        pl.kernel,
        out_type=jax.ShapeDtypeStruct((NC, N_NODES, FEAT), jnp.float32),
        mesh=mesh,
        scratch_types=[
            pltpu.VMEM((n_chunks, CHUNK), jnp.int32),
            pltpu.VMEM((n_chunks, CHUNK), jnp.int32),
            pltpu.VMEM((n_chunks, CHUNK), jnp.float32),
            pltpu.VMEM((CHUNK, FEAT), jnp.float32),
            pltpu.VMEM_SHARED((N_NODES, FEAT), jnp.float32),
            pltpu.SemaphoreType.DMA,
        ],
    )
    def k(feat_hbm, src_hbm, dst_hbm, w_hbm, out_hbm,
          src_v, dst_v, w_v, rows_v, gx_sh, sem):
        cid = lax.axis_index("c")
        sid = lax.axis_index("s")
        wid = cid * NS + sid

        zero16 = jnp.zeros((LANES,), jnp.float32)

        @pl.loop(0, CHUNK)
        def _(r):
            for s8 in range(FEAT // LANES):
                rows_v[r, pl.ds(s8 * LANES, LANES)] = zero16

        base = sid * rows_per_sub
        off = 0
        while off < rows_per_sub:
            nrows = min(CHUNK, rows_per_sub - off)
            pltpu.sync_copy(rows_v.at[pl.ds(0, nrows)],
                            gx_sh.at[pl.ds(base + off, nrows)])
            off += nrows
        plsc.subcore_barrier()

        pltpu.sync_copy(src_hbm.at[wid], src_v)
        pltpu.sync_copy(dst_hbm.at[wid], dst_v)
        pltpu.sync_copy(w_hbm.at[wid], w_v)

        @pl.loop(0, n_chunks)
        def _(j):
            pltpu.async_copy(feat_hbm.at[src_v.at[j]], rows_v, sem).wait()

            @pl.loop(0, CHUNK)
            def _(e):
                w16 = plsc.load_gather(
                    w_v, [jnp.full((LANES,), j, jnp.int32),
                          jnp.full((LANES,), e, jnp.int32)])
                for s8 in range(FEAT // LANES):
                    sl = pl.ds(s8 * LANES, LANES)
                    rows_v[e, sl] = rows_v[e, sl] * w16

            pltpu.sync_copy(rows_v, gx_sh.at[dst_v.at[j]], add=True)

        plsc.subcore_barrier()

        off = 0
        while off < rows_per_sub:
            nrows = min(CHUNK, rows_per_sub - off)
            pltpu.sync_copy(gx_sh.at[pl.ds(base + off, nrows)],
                            out_hbm.at[cid, pl.ds(base + off, nrows)])
            off += nrows

    return k(features, src_r, dst_r, w_r)


def _dense_tc(features, gx2, W1, b1, W2, b2):
    w1t = W1.T
    w2t = W2.T
    bsum = (b1 + b2).reshape(1, FEAT)
    blk = 1000

    def body(x_ref, g0_ref, g1_ref, w1_ref, w2_ref, b_ref, o_ref):
        g = g0_ref[...] + g1_ref[...]
        x = x_ref[...]
        p = jnp.dot(g + x, w1_ref[...], preferred_element_type=jnp.float32)
        p = p + jnp.dot(g * x, w2_ref[...], preferred_element_type=jnp.float32)
        p = p + b_ref[...]
        o_ref[...] = jnp.where(p >= 0, p, 0.01 * p)

    return pl.pallas_call(
        body,
        grid=(N_NODES // blk,),
        in_specs=[
            pl.BlockSpec((blk, FEAT), lambda i: (i, 0)),
            pl.BlockSpec((blk, FEAT), lambda i: (i, 0)),
            pl.BlockSpec((blk, FEAT), lambda i: (i, 0)),
            pl.BlockSpec((FEAT, FEAT), lambda i: (0, 0)),
            pl.BlockSpec((FEAT, FEAT), lambda i: (0, 0)),
            pl.BlockSpec((1, FEAT), lambda i: (0, 0)),
        ],
        out_specs=pl.BlockSpec((blk, FEAT), lambda i: (i, 0)),
        out_shape=jax.ShapeDtypeStruct((N_NODES, FEAT), jnp.float32),
    )(features, gx2[0], gx2[1], w1t, w2t, bsum)


def kernel(features, edge_index, edge_weight, W1, b1, W2, b2):
    src = edge_index[0].astype(jnp.int32)
    dst = edge_index[1].astype(jnp.int32)
    w = edge_weight.astype(jnp.float32)
    n_edges = src.shape[0]
    n_chunks = -(-n_edges // (NW * CHUNK))
    pad = NW * n_chunks * CHUNK - n_edges
    src_r = jnp.pad(src, (0, pad)).reshape(NW, n_chunks, CHUNK)
    dst_r = jnp.pad(dst, (0, pad)).reshape(NW, n_chunks, CHUNK)
    w_r = jnp.pad(w, (0, pad)).reshape(NW, n_chunks, CHUNK)
    gx2 = _spmm_sc(features, src_r, dst_r, w_r, n_chunks)
    return _dense_tc(features, gx2, W1, b1, W2, b2)


# speedup vs baseline: 3.9411x; 3.9411x over previous
import dataclasses
import functools

import jax
import jax.numpy as jnp
from jax import lax
from jax.experimental import pallas as pl
from jax.experimental.pallas import tpu as pltpu
from jax.experimental.pallas import tpu_sc as plsc

N_NODES = 10000
N_PAD = 10240
FEAT = 128
NC, NS, LANES = 2, 16, 16
NW = NC * NS
CHUNK = 128


def _sc_compiler_params():
    cp = pltpu.CompilerParams()
    if "needs_layout_passes" in pltpu.CompilerParams.__dataclass_fields__:
        cp = dataclasses.replace(cp, needs_layout_passes=False)
    return cp


def _spmm_sc(features, src_r, dst_r, w_r, n_chunks):
    mesh = plsc.VectorSubcoreMesh(core_axis_name="c", subcore_axis_name="s")
    stripe = N_PAD // NS

    @functools.partial(
        pl.kernel,
        out_type=jax.ShapeDtypeStruct((NC, N_PAD, FEAT), jnp.float32),
        mesh=mesh,
        scratch_types=[
            pltpu.VMEM((n_chunks, CHUNK), jnp.int32),
            pltpu.VMEM((n_chunks, CHUNK), jnp.int32),
            pltpu.VMEM((n_chunks, CHUNK), jnp.float32),
            pltpu.VMEM((CHUNK, FEAT), jnp.float32),
            pltpu.VMEM_SHARED((N_PAD, FEAT), jnp.float32),
            pltpu.SemaphoreType.DMA,
        ],
        compiler_params=_sc_compiler_params(),
    )
    def k(feat_hbm, src_hbm, dst_hbm, w_hbm, out_hbm,
          src_v, dst_v, w_v, rows_v, gx_sh, sem):
        cid = lax.axis_index("c")
        sid = lax.axis_index("s")
        wid = cid * NS + sid

        zero16 = jnp.zeros((LANES,), jnp.float32)

        @pl.loop(0, CHUNK)
        def _(r):
            for s8 in range(FEAT // LANES):
                rows_v[r, pl.ds(s8 * LANES, LANES)] = zero16

        base = pl.multiple_of(sid * stripe, 8)
        for off in range(0, stripe, CHUNK):
            pltpu.sync_copy(rows_v, gx_sh.at[pl.ds(base + off, CHUNK)])
        plsc.subcore_barrier()

        pltpu.sync_copy(src_hbm.at[wid], src_v)
        pltpu.sync_copy(dst_hbm.at[wid], dst_v)
        pltpu.sync_copy(w_hbm.at[wid], w_v)

        @pl.loop(0, n_chunks)
        def _(j):
            pltpu.async_copy(feat_hbm.at[src_v.at[j]], rows_v, sem).wait()

            @pl.loop(0, CHUNK)
            def _(e):
                w16 = plsc.load_gather(
                    w_v, [jnp.full((LANES,), j, jnp.int32),
                          jnp.full((LANES,), e, jnp.int32)])
                for s8 in range(FEAT // LANES):
                    sl = pl.ds(s8 * LANES, LANES)
                    rows_v[e, sl] = rows_v[e, sl] * w16

            pltpu.sync_copy(rows_v, gx_sh.at[dst_v.at[j]], add=True)

        plsc.subcore_barrier()

        pltpu.sync_copy(gx_sh.at[pl.ds(base, stripe)],
                        out_hbm.at[cid, pl.ds(base, stripe)])

    return k(features, src_r, dst_r, w_r)


def _dense_tc(features, gx2, W1, b1, W2, b2):
    w1t = W1.T
    w2t = W2.T
    bsum = (b1 + b2).reshape(1, FEAT)
    blk = 1000

    def body(x_ref, g0_ref, g1_ref, w1_ref, w2_ref, b_ref, o_ref):
        g = g0_ref[...] + g1_ref[...]
        x = x_ref[...]
        p = jnp.dot(g + x, w1_ref[...], preferred_element_type=jnp.float32)
        p = p + jnp.dot(g * x, w2_ref[...], preferred_element_type=jnp.float32)
        p = p + b_ref[...]
        o_ref[...] = jnp.where(p >= 0, p, 0.01 * p)

    return pl.pallas_call(
        body,
        grid=(N_NODES // blk,),
        in_specs=[
            pl.BlockSpec((blk, FEAT), lambda i: (i, 0)),
            pl.BlockSpec((blk, FEAT), lambda i: (i, 0)),
            pl.BlockSpec((blk, FEAT), lambda i: (i, 0)),
            pl.BlockSpec((FEAT, FEAT), lambda i: (0, 0)),
            pl.BlockSpec((FEAT, FEAT), lambda i: (0, 0)),
            pl.BlockSpec((1, FEAT), lambda i: (0, 0)),
        ],
        out_specs=pl.BlockSpec((blk, FEAT), lambda i: (i, 0)),
        out_shape=jax.ShapeDtypeStruct((N_NODES, FEAT), jnp.float32),
    )(features, gx2[0], gx2[1], w1t, w2t, bsum)


def kernel(features, edge_index, edge_weight, W1, b1, W2, b2):
    src = edge_index[0].astype(jnp.int32)
    dst = edge_index[1].astype(jnp.int32)
    w = edge_weight.astype(jnp.float32)
    n_edges = src.shape[0]
    n_chunks = -(-n_edges // (NW * CHUNK))
    pad = NW * n_chunks * CHUNK - n_edges
    src_r = jnp.pad(src, (0, pad)).reshape(NW, n_chunks, CHUNK)
    dst_r = jnp.pad(dst, (0, pad)).reshape(NW, n_chunks, CHUNK)
    w_r = jnp.pad(w, (0, pad)).reshape(NW, n_chunks, CHUNK)
    gx2 = _spmm_sc(features, src_r, dst_r, w_r, n_chunks)
    return _dense_tc(features, gx2, W1, b1, W2, b2)
